# Initial kernel scaffold; baseline (speedup 1.0000x reference)
import jax
import jax.numpy as jnp
from jax.experimental import pallas as pl


def kernel(x, edge_index, W1, b1, W2, b2, W3, b3, Wv, bv):
    raise NotImplementedError("write your pallas kernel here")


# speedup vs baseline: 34.7782x; 34.7782x over previous
import functools

import jax
import jax.numpy as jnp
from jax import lax
from jax.experimental import pallas as pl
from jax.experimental.pallas import tpu as pltpu
from jax.experimental.pallas import tpu_sc as plsc

N_NODES = 100000
N_EDGES = 6400000
IN_DIM = 8
HIDDEN = 128
XCOLS = 16

NUM_WORKERS = 32
ROW = 128
ROWS_PER_ITER = 8
ITERS = 196
EDGES_PER_TILE = ROW * ROWS_PER_ITER * ITERS
E_PAD = EDGES_PER_TILE * NUM_WORKERS
AGG_ROWS = 102400
ZROWS = AGG_ROWS // 16


def _sc_body(xaug_hbm, src_hbm, dst_hbm, zeros_hbm, out_hbm,
             src_v, dst_v, rows_v, agg_sh, sem):
    c = lax.axis_index("c")
    s = lax.axis_index("s")
    w = c * 16 + s

    pltpu.sync_copy(zeros_hbm, agg_sh.at[pl.ds(s * ZROWS, ZROWS)])
    plsc.subcore_barrier()

    base_row = w * (ROWS_PER_ITER * ITERS)

    @pl.loop(0, ITERS)
    def _edge_iter(i):
        r0 = base_row + i * ROWS_PER_ITER
        pltpu.sync_copy(src_hbm.at[pl.ds(r0, ROWS_PER_ITER)], src_v)
        pltpu.sync_copy(dst_hbm.at[pl.ds(r0, ROWS_PER_ITER)], dst_v)
        descs = []
        for j in range(ROWS_PER_ITER):
            descs.append(pltpu.async_copy(
                xaug_hbm.at[dst_v.at[j]],
                rows_v.at[pl.ds(j * ROW, ROW)], sem))
        for d in descs:
            d.wait()
        for j in range(ROWS_PER_ITER):
            pltpu.sync_copy(rows_v.at[pl.ds(j * ROW, ROW)],
                            agg_sh.at[src_v.at[j]], add=True)

    plsc.subcore_barrier()
    pltpu.sync_copy(agg_sh.at[pl.ds(s * ZROWS, ZROWS)],
                    out_hbm.at[c].at[pl.ds(s * ZROWS, ZROWS)])


def _scatter_parts(xaug, src2d, dst2d, zeros_hbm):
    mesh = plsc.VectorSubcoreMesh(core_axis_name="c", subcore_axis_name="s")
    f = pl.kernel(
        _sc_body,
        out_type=jax.ShapeDtypeStruct((2, AGG_ROWS, XCOLS), jnp.float32),
        mesh=mesh,
        scratch_types=[
            pltpu.VMEM((ROWS_PER_ITER, ROW), jnp.int32),
            pltpu.VMEM((ROWS_PER_ITER, ROW), jnp.int32),
            pltpu.VMEM((ROWS_PER_ITER * ROW, XCOLS), jnp.float32),
            pltpu.VMEM_SHARED((AGG_ROWS, XCOLS), jnp.float32),
            pltpu.SemaphoreType.DMA,
        ],
        compiler_params=pltpu.CompilerParams(use_tc_tiling_on_sc=False),
    )
    return f(xaug, src2d, dst2d, zeros_hbm)


NBLK = 50
BLK = N_NODES // NBLK


def _mlp_body(parts_ref, x_ref, w1t_ref, w1p_ref, b1_ref, w2t_ref, b2_ref,
              w3t_ref, b3_ref, wv_ref, bv_ref, out_ref, acc_ref):
    i = pl.program_id(0)
    p = parts_ref[0] + parts_ref[1]
    lane = lax.broadcasted_iota(jnp.int32, (BLK, XCOLS), 1)
    deg = jnp.sum(jnp.where(lane == IN_DIM, p, 0.0), axis=1, keepdims=True)
    deg = jnp.maximum(deg, 1.0)
    aggw = lax.dot_general(p, w1p_ref[...],
                           (((1,), (0,)), ((), ()))) / deg
    xw = lax.dot_general(x_ref[...], w1t_ref[...], (((1,), (0,)), ((), ())))
    h = jnp.maximum(xw + aggw + b1_ref[...], 0.0)
    h = jnp.maximum(lax.dot_general(h, w2t_ref[...], (((1,), (0,)), ((), ())))
                    + b2_ref[...], 0.0)
    h = jnp.maximum(lax.dot_general(h, w3t_ref[...], (((1,), (0,)), ((), ())))
                    + b3_ref[...], 0.0)
    part_sum = jnp.sum(h, axis=0, keepdims=True)

    @pl.when(i == 0)
    def _():
        acc_ref[...] = part_sum

    @pl.when(i > 0)
    def _():
        acc_ref[...] = acc_ref[...] + part_sum

    @pl.when(i == NBLK - 1)
    def _():
        m = acc_ref[...] / jnp.float32(N_NODES)
        v = jnp.sum(m * wv_ref[...], axis=1, keepdims=True) + bv_ref[...]
        out_ref[...] = jnp.tanh(v)


def _mlp(parts, x, w1t, w1p, b1, w2t, b2, w3t, b3, wv, bv):
    return pl.pallas_call(
        _mlp_body,
        grid=(NBLK,),
        in_specs=[
            pl.BlockSpec((2, BLK, XCOLS), lambda i: (0, i, 0)),
            pl.BlockSpec((BLK, IN_DIM), lambda i: (i, 0)),
            pl.BlockSpec((IN_DIM, HIDDEN), lambda i: (0, 0)),
            pl.BlockSpec((XCOLS, HIDDEN), lambda i: (0, 0)),
            pl.BlockSpec((1, HIDDEN), lambda i: (0, 0)),
            pl.BlockSpec((HIDDEN, HIDDEN), lambda i: (0, 0)),
            pl.BlockSpec((1, HIDDEN), lambda i: (0, 0)),
            pl.BlockSpec((HIDDEN, HIDDEN), lambda i: (0, 0)),
            pl.BlockSpec((1, HIDDEN), lambda i: (0, 0)),
            pl.BlockSpec((1, HIDDEN), lambda i: (0, 0)),
            pl.BlockSpec((1, 1), lambda i: (0, 0)),
        ],
        out_specs=pl.BlockSpec((1, 1), lambda i: (0, 0)),
        out_shape=jax.ShapeDtypeStruct((1, 1), jnp.float32),
        scratch_shapes=[pltpu.VMEM((1, HIDDEN), jnp.float32)],
        compiler_params=pltpu.CompilerParams(
            dimension_semantics=("arbitrary",)),
    )(parts, x, w1t, w1p, b1, w2t, b2, w3t, b3, wv, bv)


def kernel(x, edge_index, W1, b1, W2, b2, W3, b3, Wv, bv):
    src = edge_index[0]
    dst = edge_index[1]
    pad = E_PAD - N_EDGES
    src_p = jnp.concatenate(
        [src, jnp.full((pad,), N_NODES, dtype=jnp.int32)]).reshape(-1, ROW)
    dst_p = jnp.concatenate(
        [dst, jnp.zeros((pad,), dtype=jnp.int32)]).reshape(-1, ROW)
    xaug = jnp.zeros((N_NODES, XCOLS), dtype=jnp.float32)
    xaug = xaug.at[:, 0:IN_DIM].set(x).at[:, IN_DIM].set(1.0)
    zeros_hbm = jnp.zeros((ZROWS, XCOLS), dtype=jnp.float32)

    parts = _scatter_parts(xaug, src_p, dst_p, zeros_hbm)

    w1p = jnp.zeros((XCOLS, HIDDEN), dtype=jnp.float32).at[0:IN_DIM].set(W1.T)
    out = _mlp(parts, x, W1.T, w1p, b1.reshape(1, -1), W2.T,
               b2.reshape(1, -1), W3.T, b3.reshape(1, -1), Wv.reshape(1, -1),
               bv.reshape(1, 1))
    return jnp.squeeze(out)


# speedup vs baseline: 42.6493x; 1.2263x over previous
import functools

import jax
import jax.numpy as jnp
from jax import lax
from jax.experimental import pallas as pl
from jax.experimental.pallas import tpu as pltpu
from jax.experimental.pallas import tpu_sc as plsc

N_NODES = 100000
N_EDGES = 6400000
IN_DIM = 8
HIDDEN = 128
XCOLS = 16

NUM_WORKERS = 32
ROW = 128
ROWS_PER_ITER = 8
EDGE_ROWS = N_EDGES // ROW
N_CHUNKS = EDGE_ROWS // ROWS_PER_ITER
CHUNKS_MAIN = 196
CHUNKS_LAST = N_CHUNKS - 31 * CHUNKS_MAIN
AGG_ROWS = 102400
ZROWS = AGG_ROWS // 16


def _sc_body(xaug_hbm, ei_hbm, zeros_hbm, out_hbm,
             idx_v0, rows_v0, agg_sh, gsem0):
    c = lax.axis_index("c")
    s = lax.axis_index("s")
    w = c * 16 + s

    pltpu.sync_copy(zeros_hbm, agg_sh.at[pl.ds(s * ZROWS, ZROWS)])
    plsc.subcore_barrier()

    base_chunk = w * CHUNKS_MAIN
    n = jnp.where(w == NUM_WORKERS - 1, CHUNKS_LAST, CHUNKS_MAIN)

    def load_idx(i, idx_v):
        r0 = (base_chunk + i) * ROWS_PER_ITER
        pltpu.sync_copy(ei_hbm.at[:, pl.ds(r0, ROWS_PER_ITER)], idx_v)

    def fire_gathers(idx_v, rows_v, gsem):
        for j in range(ROWS_PER_ITER):
            pltpu.async_copy(xaug_hbm.at[idx_v.at[1].at[j]],
                             rows_v.at[pl.ds(j * ROW, ROW)], gsem)

    def drain_gathers(idx_v, rows_v, gsem):
        for j in range(ROWS_PER_ITER):
            pltpu.make_async_copy(xaug_hbm.at[idx_v.at[1].at[j]],
                                  rows_v.at[pl.ds(j * ROW, ROW)], gsem).wait()

    def fire_scatters(idx_v, rows_v, ssem):
        for j in range(ROWS_PER_ITER):
            pltpu.async_copy(rows_v.at[pl.ds(j * ROW, ROW)],
                             agg_sh.at[idx_v.at[0].at[j]], ssem, add=True)

    def drain_scatters(idx_v, rows_v, ssem):
        for j in range(ROWS_PER_ITER):
            pltpu.make_async_copy(rows_v.at[pl.ds(j * ROW, ROW)],
                                  agg_sh.at[idx_v.at[0].at[j]], ssem).wait()

    @pl.loop(0, n)
    def _edge_iter(i):
        load_idx(i, idx_v0)
        fire_gathers(idx_v0, rows_v0, gsem0)
        drain_gathers(idx_v0, rows_v0, gsem0)
        for j in range(ROWS_PER_ITER):
            pltpu.sync_copy(rows_v0.at[pl.ds(j * ROW, ROW)],
                            agg_sh.at[idx_v0.at[0].at[j]], add=True)

    plsc.subcore_barrier()
    pltpu.sync_copy(agg_sh.at[pl.ds(s * ZROWS, ZROWS)],
                    out_hbm.at[c].at[pl.ds(s * ZROWS, ZROWS)])


def _scatter_parts(xaug, ei3, zeros_hbm):
    mesh = plsc.VectorSubcoreMesh(core_axis_name="c", subcore_axis_name="s")
    f = pl.kernel(
        _sc_body,
        out_type=jax.ShapeDtypeStruct((2, AGG_ROWS, XCOLS), jnp.float32),
        mesh=mesh,
        scratch_types=[
            pltpu.VMEM((2, ROWS_PER_ITER, ROW), jnp.int32),
            pltpu.VMEM((ROWS_PER_ITER * ROW, XCOLS), jnp.float32),
            pltpu.VMEM_SHARED((AGG_ROWS, XCOLS), jnp.float32),
            pltpu.SemaphoreType.DMA,
        ],
        compiler_params=pltpu.CompilerParams(use_tc_tiling_on_sc=False),
    )
    return f(xaug, ei3, zeros_hbm)


NBLK = 50
BLK = N_NODES // NBLK


def _mlp_body(parts_ref, x_ref, w1t_ref, w1p_ref, b1_ref, w2t_ref, b2_ref,
              w3t_ref, b3_ref, wv_ref, bv_ref, out_ref, acc_ref):
    i = pl.program_id(0)
    p = parts_ref[0] + parts_ref[1]
    lane = lax.broadcasted_iota(jnp.int32, (BLK, XCOLS), 1)
    deg = jnp.sum(jnp.where(lane == IN_DIM, p, 0.0), axis=1, keepdims=True)
    deg = jnp.maximum(deg, 1.0)
    aggw = lax.dot_general(p, w1p_ref[...],
                           (((1,), (0,)), ((), ()))) / deg
    xw = lax.dot_general(x_ref[...], w1t_ref[...], (((1,), (0,)), ((), ())))
    h = jnp.maximum(xw + aggw + b1_ref[...], 0.0)
    h = jnp.maximum(lax.dot_general(h, w2t_ref[...], (((1,), (0,)), ((), ())))
                    + b2_ref[...], 0.0)
    h = jnp.maximum(lax.dot_general(h, w3t_ref[...], (((1,), (0,)), ((), ())))
                    + b3_ref[...], 0.0)
    part_sum = jnp.sum(h, axis=0, keepdims=True)

    @pl.when(i == 0)
    def _():
        acc_ref[...] = part_sum

    @pl.when(i > 0)
    def _():
        acc_ref[...] = acc_ref[...] + part_sum

    @pl.when(i == NBLK - 1)
    def _():
        m = acc_ref[...] / jnp.float32(N_NODES)
        v = jnp.sum(m * wv_ref[...], axis=1, keepdims=True) + bv_ref[...]
        out_ref[...] = jnp.tanh(v)


def _mlp(parts, x, w1t, w1p, b1, w2t, b2, w3t, b3, wv, bv):
    return pl.pallas_call(
        _mlp_body,
        grid=(NBLK,),
        in_specs=[
            pl.BlockSpec((2, BLK, XCOLS), lambda i: (0, i, 0)),
            pl.BlockSpec((BLK, IN_DIM), lambda i: (i, 0)),
            pl.BlockSpec((IN_DIM, HIDDEN), lambda i: (0, 0)),
            pl.BlockSpec((XCOLS, HIDDEN), lambda i: (0, 0)),
            pl.BlockSpec((1, HIDDEN), lambda i: (0, 0)),
            pl.BlockSpec((HIDDEN, HIDDEN), lambda i: (0, 0)),
            pl.BlockSpec((1, HIDDEN), lambda i: (0, 0)),
            pl.BlockSpec((HIDDEN, HIDDEN), lambda i: (0, 0)),
            pl.BlockSpec((1, HIDDEN), lambda i: (0, 0)),
            pl.BlockSpec((1, HIDDEN), lambda i: (0, 0)),
            pl.BlockSpec((1, 1), lambda i: (0, 0)),
        ],
        out_specs=pl.BlockSpec((1, 1), lambda i: (0, 0)),
        out_shape=jax.ShapeDtypeStruct((1, 1), jnp.float32),
        scratch_shapes=[pltpu.VMEM((1, HIDDEN), jnp.float32)],
        compiler_params=pltpu.CompilerParams(
            dimension_semantics=("arbitrary",)),
    )(parts, x, w1t, w1p, b1, w2t, b2, w3t, b3, wv, bv)


def kernel(x, edge_index, W1, b1, W2, b2, W3, b3, Wv, bv):
    ei3 = edge_index.reshape(2, EDGE_ROWS, ROW)
    xaug = jnp.zeros((N_NODES, XCOLS), dtype=jnp.float32)
    xaug = xaug.at[:, 0:IN_DIM].set(x).at[:, IN_DIM].set(1.0)
    zeros_hbm = jnp.zeros((ZROWS, XCOLS), dtype=jnp.float32)

    parts = _scatter_parts(xaug, ei3, zeros_hbm)

    w1p = jnp.zeros((XCOLS, HIDDEN), dtype=jnp.float32).at[0:IN_DIM].set(W1.T)
    out = _mlp(parts, x, W1.T, w1p, b1.reshape(1, -1), W2.T,
               b2.reshape(1, -1), W3.T, b3.reshape(1, -1), Wv.reshape(1, -1),
               bv.reshape(1, 1))
    return jnp.squeeze(out)
